# baseline (speedup 1.0000x reference)
import functools

import jax
import jax.numpy as jnp
from jax import lax
from jax.experimental import pallas as pl
from jax.experimental.pallas import tpu as pltpu
from jax.experimental.pallas import tpu_sc as plsc

D = 32
B = 16384
NC = 2
NS = 16
NW = NC * NS
BPW = B // NW
L = 16
CHUNK = 128
NCHUNK = BPW // CHUNK


def _sc_body(uid_hbm, iid_hbm, ut_hbm, it_hbm, ubt_hbm, ibt_hbm, gb_hbm,
             out_hbm,
             idx_u, idx_i, u_rows, i_rows, ub, ib, gb, out_v, sem):
    wid = lax.axis_index("s") * NC + lax.axis_index("c")
    base = wid * BPW

    pltpu.sync_copy(uid_hbm.at[pl.ds(base, BPW)], idx_u)
    pltpu.sync_copy(iid_hbm.at[pl.ds(base, BPW)], idx_i)
    pltpu.sync_copy(gb_hbm, gb.at[pl.ds(0, 1)])

    copies = []
    for c in range(NCHUNK):
        sl = pl.ds(c * CHUNK, CHUNK)
        copies.append(pltpu.async_copy(
            ut_hbm.at[idx_u.at[sl]], u_rows.at[sl], sem))
        copies.append(pltpu.async_copy(
            it_hbm.at[idx_i.at[sl]], i_rows.at[sl], sem))
        copies.append(pltpu.async_copy(ubt_hbm.at[idx_u.at[sl]], ub.at[sl], sem))
        copies.append(pltpu.async_copy(ibt_hbm.at[idx_i.at[sl]], ib.at[sl], sem))
    for cp in copies:
        cp.wait()

    iota = lax.broadcasted_iota(jnp.int32, (L,), 0)
    gbias = gb[pl.ds(0, L)][0]

    def group(g, carry):
        rows = iota + g * L
        acc = ub[pl.ds(g * L, L)] + ib[pl.ds(g * L, L)] + gbias
        for d in range(D):
            col = jnp.full((L,), d, jnp.int32)
            uvec = plsc.load_gather(u_rows, [rows, col])
            ivec = plsc.load_gather(i_rows, [rows, col])
            acc = acc + uvec * ivec
        out_v[pl.ds(g * L, L)] = acc
        return carry

    lax.fori_loop(0, BPW // L, group, 0)

    pltpu.sync_copy(out_v, out_hbm.at[pl.ds(base, BPW)])


@jax.jit
def kernel(user_id, item_id, user_table, item_table, user_bias_table,
           item_bias_table, global_bias):
    user_id = user_id.astype(jnp.int32)
    item_id = item_id.astype(jnp.int32)
    user_bias_table = user_bias_table.reshape(-1)
    item_bias_table = item_bias_table.reshape(-1)
    mesh = plsc.VectorSubcoreMesh(core_axis_name="c", subcore_axis_name="s")
    f = pl.kernel(
        _sc_body,
        out_type=jax.ShapeDtypeStruct((B,), jnp.float32),
        mesh=mesh,
        scratch_types=[
            pltpu.VMEM((BPW,), jnp.int32),
            pltpu.VMEM((BPW,), jnp.int32),
            pltpu.VMEM((BPW, D), jnp.float32),
            pltpu.VMEM((BPW, D), jnp.float32),
            pltpu.VMEM((BPW,), jnp.float32),
            pltpu.VMEM((BPW,), jnp.float32),
            pltpu.VMEM((L,), jnp.float32),
            pltpu.VMEM((BPW,), jnp.float32),
            pltpu.SemaphoreType.DMA,
        ],
        compiler_params=pltpu.CompilerParams(
            needs_layout_passes=False, use_tc_tiling_on_sc=False),
    )
    return f(user_id, item_id, user_table, item_table, user_bias_table,
             item_bias_table, global_bias)
